# baseline (speedup 1.0000x reference)
import functools

import jax
import jax.numpy as jnp
from jax import lax
from jax.experimental import pallas as pl
from jax.experimental.pallas import tpu as pltpu
from jax.experimental.pallas import tpu_sc as plsc

B = 16384
EMB = 64
HID = 192
CAT = EMB * 4
EPS = 1e-5

NC = 2
NS = 16
NW = NC * NS
BPW = B // NW
CG = 128
NCH = BPW // CG


def _sc_gather_build():
    mesh = plsc.VectorSubcoreMesh(core_axis_name="c", subcore_axis_name="s")
    part = jax.ShapeDtypeStruct((B, EMB), jnp.float32)

    @functools.partial(
        pl.kernel,
        mesh=mesh,
        compiler_params=pltpu.CompilerParams(use_tc_tiling_on_sc=False),
        out_type=[part, part, part, part],
        scratch_types=[
            pltpu.VMEM((CG,), jnp.int32),
            pltpu.VMEM((CG, EMB), jnp.float32),
            pltpu.SemaphoreType.DMA,
        ],
    )
    def sc_gather(t0, t1, t2, t3, i0, i1, i2, i3,
                  o0, o1, o2, o3, idx_v, rows_v, sem):
        wid = lax.axis_index("s") * NC + lax.axis_index("c")
        base = wid * BPW
        for tab, idx, out in ((t0, i0, o0), (t1, i1, o1),
                              (t2, i2, o2), (t3, i3, o3)):
            for c in range(NCH):
                off = base + c * CG
                pltpu.sync_copy(idx.at[pl.ds(off, CG)], idx_v)
                pltpu.async_copy(tab.at[idx_v], rows_v, sem).wait()
                pltpu.sync_copy(rows_v, out.at[pl.ds(off, CG), :])

    return sc_gather


_sc_gather = _sc_gather_build()


BR = 2048
NB = B // BR


def _tc_stats_body(x0_ref, x1_ref, x2_ref, x3_ref, s_ref):
    @pl.when(pl.program_id(0) == 0)
    def _():
        s_ref[...] = jnp.zeros_like(s_ref)

    for t, x_ref in enumerate((x0_ref, x1_ref, x2_ref, x3_ref)):
        x = x_ref[...]
        part = jnp.stack([jnp.sum(x, axis=0), jnp.sum(x * x, axis=0)])
        s_ref[:, t * EMB:(t + 1) * EMB] += part


def _tc_apply_body(s_ref, x0_ref, x1_ref, x2_ref, x3_ref, gamma_ref,
                   beta_ref, g_ref, v_ref, bias_ref, o_ref):
    mean = s_ref[0] * (1.0 / B)
    var = s_ref[1] * (1.0 / B) - mean * mean
    scale = gamma_ref[0] * lax.rsqrt(var + EPS)
    shift = beta_ref[0] - mean * scale
    v = v_ref[...]
    w = v * (g_ref[...] * lax.rsqrt(jnp.sum(v * v, axis=1)))[:, None]
    bias_eff = bias_ref[...] + jnp.sum(w * shift[None, :], axis=1)
    acc = jnp.zeros((BR, HID), jnp.float32)
    for t, x_ref in enumerate((x0_ref, x1_ref, x2_ref, x3_ref)):
        ws = w[:, t * EMB:(t + 1) * EMB] * scale[None, t * EMB:(t + 1) * EMB]
        acc = acc + lax.dot_general(
            x_ref[...], ws, (((1,), (1,)), ((), ())),
            preferred_element_type=jnp.float32,
        )
    o_ref[...] = jax.nn.sigmoid(acc + bias_eff[None, :])


def kernel(last_test, last_question, last_tag, last_qclass,
           emb_test, emb_question, emb_tag, emb_qclass,
           bn_gamma, bn_beta, wn_g, wn_v, bias):
    i_test = last_test.astype(jnp.int32)
    i_question = last_question.astype(jnp.int32)
    i_tag = last_tag.astype(jnp.int32)
    i_qclass = last_qclass.astype(jnp.int32)

    x_test, x_question, x_tag, x_qclass = _sc_gather(
        emb_test, emb_question, emb_tag, emb_qclass,
        i_test, i_question, i_tag, i_qclass,
    )
    xs = (x_test, x_question, x_tag, x_qclass)

    xblk = pl.BlockSpec((BR, EMB), lambda i: (i, 0))
    stats = pl.pallas_call(
        _tc_stats_body,
        grid=(NB,),
        in_specs=[xblk] * 4,
        out_specs=pl.BlockSpec((2, CAT), lambda i: (0, 0)),
        out_shape=jax.ShapeDtypeStruct((2, CAT), jnp.float32),
    )(*xs)

    full = lambda *s: pl.BlockSpec(s, lambda i: (0,) * len(s))
    return pl.pallas_call(
        _tc_apply_body,
        grid=(NB,),
        in_specs=[full(2, CAT), xblk, xblk, xblk, xblk,
                  full(1, CAT), full(1, CAT), full(HID),
                  full(HID, CAT), full(HID)],
        out_specs=pl.BlockSpec((BR, HID), lambda i: (i, 0)),
        out_shape=jax.ShapeDtypeStruct((B, HID), jnp.float32),
    )(stats, *xs, bn_gamma.reshape(1, CAT), bn_beta.reshape(1, CAT),
      wn_g, wn_v, bias)


# speedup vs baseline: 1.5978x; 1.5978x over previous
import functools

import jax
import jax.numpy as jnp
from jax import lax
from jax.experimental import pallas as pl
from jax.experimental.pallas import tpu as pltpu
from jax.experimental.pallas import tpu_sc as plsc

B = 16384
EMB = 64
HID = 192
CAT = EMB * 4
EPS = 1e-5
N_TAG = 1000
N_CLASS = 1000

NC = 2
NS = 16
NW = NC * NS
BPW = B // NW
CGB = 64
NCHB = BPW // CGB
L = 16

BLK = 1024
NBLK = B // BLK

BR = 2048
NB = B // BR


def _sc_gather_big_build():
    mesh = plsc.VectorSubcoreMesh(core_axis_name="c", subcore_axis_name="s")
    part = jax.ShapeDtypeStruct((B, EMB), jnp.float32)

    @functools.partial(
        pl.kernel,
        mesh=mesh,
        compiler_params=pltpu.CompilerParams(needs_layout_passes=False),
        out_type=[part, part],
        scratch_types=[
            pltpu.VMEM((CGB,), jnp.int32),
            pltpu.VMEM((CGB, EMB), jnp.float32),
            pltpu.SemaphoreType.DMA,
        ],
    )
    def body(t0, t1, i0, i1, o0, o1, idx_v, rows_v, sem):
        wid = lax.axis_index("s") * NC + lax.axis_index("c")
        base = wid * BPW
        for tab, idx, out in ((t0, i0, o0), (t1, i1, o1)):
            def chunk(c, carry):
                off = base + c * CGB
                pltpu.sync_copy(idx.at[pl.ds(off, CGB)], idx_v)
                descs = []
                for j in range(CGB // L):
                    v = idx_v[pl.ds(j * L, L)]
                    for l in range(L):
                        descs.append(pltpu.async_copy(
                            tab.at[pl.ds(v[l], 1), :],
                            rows_v.at[pl.ds(j * L + l, 1), :], sem))
                for d in descs:
                    d.wait()
                pltpu.sync_copy(rows_v, out.at[pl.ds(off, CGB), :])
                return carry

            lax.fori_loop(0, NCHB, chunk, 0, unroll=False)

    return body


_sc_gather_big = _sc_gather_big_build()


def _onehot_body(it_ref, iq_ref, tt_ref, tq_ref, ot_ref, oq_ref):
    rows = lax.broadcasted_iota(jnp.int32, (BLK, N_TAG), 1)
    oh = (it_ref[...][:, None] == rows).astype(jnp.float32)
    ot_ref[...] = jnp.dot(oh, tt_ref[...], preferred_element_type=jnp.float32)
    oh = (iq_ref[...][:, None] == rows).astype(jnp.float32)
    oq_ref[...] = jnp.dot(oh, tq_ref[...], preferred_element_type=jnp.float32)


def _tc_stats_body(x0_ref, x1_ref, x2_ref, x3_ref, s_ref):
    @pl.when(pl.program_id(0) == 0)
    def _():
        s_ref[...] = jnp.zeros_like(s_ref)

    for t, x_ref in enumerate((x0_ref, x1_ref, x2_ref, x3_ref)):
        x = x_ref[...]
        part = jnp.stack([jnp.sum(x, axis=0), jnp.sum(x * x, axis=0)])
        s_ref[:, t * EMB:(t + 1) * EMB] += part


def _tc_apply_body(s_ref, x0_ref, x1_ref, x2_ref, x3_ref, gamma_ref,
                   beta_ref, g_ref, v_ref, bias_ref, o_ref):
    mean = s_ref[0] * (1.0 / B)
    var = s_ref[1] * (1.0 / B) - mean * mean
    scale = gamma_ref[0] * lax.rsqrt(var + EPS)
    shift = beta_ref[0] - mean * scale
    v = v_ref[...]
    w = v * (g_ref[...] * lax.rsqrt(jnp.sum(v * v, axis=1)))[:, None]
    bias_eff = bias_ref[...] + jnp.sum(w * shift[None, :], axis=1)
    acc = jnp.zeros((BR, HID), jnp.float32)
    for t, x_ref in enumerate((x0_ref, x1_ref, x2_ref, x3_ref)):
        ws = w[:, t * EMB:(t + 1) * EMB] * scale[None, t * EMB:(t + 1) * EMB]
        acc = acc + lax.dot_general(
            x_ref[...], ws, (((1,), (1,)), ((), ())),
            preferred_element_type=jnp.float32,
        )
    o_ref[...] = jax.nn.sigmoid(acc + bias_eff[None, :])


def kernel(last_test, last_question, last_tag, last_qclass,
           emb_test, emb_question, emb_tag, emb_qclass,
           bn_gamma, bn_beta, wn_g, wn_v, bias):
    i_test = last_test.astype(jnp.int32)
    i_question = last_question.astype(jnp.int32)
    i_tag = last_tag.astype(jnp.int32)
    i_qclass = last_qclass.astype(jnp.int32)

    x_test, x_question = _sc_gather_big(
        emb_test, emb_question, i_test, i_question,
    )

    x_tag, x_qclass = pl.pallas_call(
        _onehot_body,
        grid=(NBLK,),
        in_specs=[
            pl.BlockSpec((BLK,), lambda i: (i,)),
            pl.BlockSpec((BLK,), lambda i: (i,)),
            pl.BlockSpec((N_TAG, EMB), lambda i: (0, 0)),
            pl.BlockSpec((N_CLASS, EMB), lambda i: (0, 0)),
        ],
        out_specs=[
            pl.BlockSpec((BLK, EMB), lambda i: (i, 0)),
            pl.BlockSpec((BLK, EMB), lambda i: (i, 0)),
        ],
        out_shape=[
            jax.ShapeDtypeStruct((B, EMB), jnp.float32),
            jax.ShapeDtypeStruct((B, EMB), jnp.float32),
        ],
    )(i_tag, i_qclass, emb_tag, emb_qclass)

    xs = (x_test, x_question, x_tag, x_qclass)

    xblk = pl.BlockSpec((BR, EMB), lambda i: (i, 0))
    stats = pl.pallas_call(
        _tc_stats_body,
        grid=(NB,),
        in_specs=[xblk] * 4,
        out_specs=pl.BlockSpec((2, CAT), lambda i: (0, 0)),
        out_shape=jax.ShapeDtypeStruct((2, CAT), jnp.float32),
    )(*xs)

    full = lambda *s: pl.BlockSpec(s, lambda i: (0,) * len(s))
    return pl.pallas_call(
        _tc_apply_body,
        grid=(NB,),
        in_specs=[full(2, CAT), xblk, xblk, xblk, xblk,
                  full(1, CAT), full(1, CAT), full(HID),
                  full(HID, CAT), full(HID)],
        out_specs=pl.BlockSpec((BR, HID), lambda i: (i, 0)),
        out_shape=jax.ShapeDtypeStruct((B, HID), jnp.float32),
    )(stats, *xs, bn_gamma.reshape(1, CAT), bn_beta.reshape(1, CAT),
      wn_g, wn_v, bias)


# speedup vs baseline: 1.6045x; 1.0042x over previous
import functools

import jax
import jax.numpy as jnp
from jax import lax
from jax.experimental import pallas as pl
from jax.experimental.pallas import tpu as pltpu
from jax.experimental.pallas import tpu_sc as plsc

B = 16384
EMB = 64
HID = 192
CAT = EMB * 4
EPS = 1e-5
N_TAG = 1000
N_CLASS = 1000

NC = 2
NS = 16
NW = NC * NS
BPW = B // NW
CGB = 64
NCHB = BPW // CGB
L = 16

BLK = 1024
NBLK = B // BLK

BR = 2048
NB = B // BR


def _sc_gather_big_build():
    mesh = plsc.VectorSubcoreMesh(core_axis_name="c", subcore_axis_name="s")
    part = jax.ShapeDtypeStruct((B, EMB), jnp.float32)

    @functools.partial(
        pl.kernel,
        mesh=mesh,
        compiler_params=pltpu.CompilerParams(
            needs_layout_passes=False, use_tc_tiling_on_sc=True),
        out_type=[part, part],
        scratch_types=[
            pltpu.VMEM((CGB,), jnp.int32),
            pltpu.VMEM((CGB, EMB), jnp.float32),
            pltpu.SemaphoreType.DMA,
        ],
    )
    def body(t0, t1, i0, i1, o0, o1, idx_v, rows_v, sem):
        wid = lax.axis_index("s") * NC + lax.axis_index("c")
        base = wid * BPW
        for tab, idx, out in ((t0, i0, o0), (t1, i1, o1)):
            def chunk(c, carry):
                off = base + c * CGB
                pltpu.sync_copy(idx.at[pl.ds(off, CGB)], idx_v)
                descs = []
                for j in range(CGB // L):
                    v = idx_v[pl.ds(j * L, L)]
                    for l in range(L):
                        descs.append(pltpu.async_copy(
                            tab.at[pl.ds(v[l], 1), :],
                            rows_v.at[pl.ds(j * L + l, 1), :], sem))
                for d in descs:
                    d.wait()
                pltpu.sync_copy(rows_v, out.at[pl.ds(off, CGB), :])
                return carry

            lax.fori_loop(0, NCHB, chunk, 0, unroll=False)

    return body


_sc_gather_big = _sc_gather_big_build()


def _onehot_body(it_ref, iq_ref, tt_ref, tq_ref, ot_ref, oq_ref):
    rows = lax.broadcasted_iota(jnp.int32, (BLK, N_TAG), 1)
    oh = (it_ref[...][:, None] == rows).astype(jnp.float32)
    ot_ref[...] = jnp.dot(oh, tt_ref[...], preferred_element_type=jnp.float32)
    oh = (iq_ref[...][:, None] == rows).astype(jnp.float32)
    oq_ref[...] = jnp.dot(oh, tq_ref[...], preferred_element_type=jnp.float32)


def _tc_stats_body(x0_ref, x1_ref, x2_ref, x3_ref, s_ref):
    @pl.when(pl.program_id(0) == 0)
    def _():
        s_ref[...] = jnp.zeros_like(s_ref)

    for t, x_ref in enumerate((x0_ref, x1_ref, x2_ref, x3_ref)):
        x = x_ref[...]
        part = jnp.stack([jnp.sum(x, axis=0), jnp.sum(x * x, axis=0)])
        s_ref[:, t * EMB:(t + 1) * EMB] += part


def _tc_apply_body(s_ref, x0_ref, x1_ref, x2_ref, x3_ref, gamma_ref,
                   beta_ref, g_ref, v_ref, bias_ref, o_ref):
    mean = s_ref[0] * (1.0 / B)
    var = s_ref[1] * (1.0 / B) - mean * mean
    scale = gamma_ref[0] * lax.rsqrt(var + EPS)
    shift = beta_ref[0] - mean * scale
    v = v_ref[...]
    w = v * (g_ref[...] * lax.rsqrt(jnp.sum(v * v, axis=1)))[:, None]
    bias_eff = bias_ref[...] + jnp.sum(w * shift[None, :], axis=1)
    acc = jnp.zeros((BR, HID), jnp.float32)
    for t, x_ref in enumerate((x0_ref, x1_ref, x2_ref, x3_ref)):
        ws = w[:, t * EMB:(t + 1) * EMB] * scale[None, t * EMB:(t + 1) * EMB]
        acc = acc + lax.dot_general(
            x_ref[...], ws, (((1,), (1,)), ((), ())),
            preferred_element_type=jnp.float32,
        )
    o_ref[...] = jax.nn.sigmoid(acc + bias_eff[None, :])


def kernel(last_test, last_question, last_tag, last_qclass,
           emb_test, emb_question, emb_tag, emb_qclass,
           bn_gamma, bn_beta, wn_g, wn_v, bias):
    i_test = last_test.astype(jnp.int32)
    i_question = last_question.astype(jnp.int32)
    i_tag = last_tag.astype(jnp.int32)
    i_qclass = last_qclass.astype(jnp.int32)

    x_test, x_question = _sc_gather_big(
        emb_test, emb_question, i_test, i_question,
    )

    x_tag, x_qclass = pl.pallas_call(
        _onehot_body,
        grid=(NBLK,),
        in_specs=[
            pl.BlockSpec((BLK,), lambda i: (i,)),
            pl.BlockSpec((BLK,), lambda i: (i,)),
            pl.BlockSpec((N_TAG, EMB), lambda i: (0, 0)),
            pl.BlockSpec((N_CLASS, EMB), lambda i: (0, 0)),
        ],
        out_specs=[
            pl.BlockSpec((BLK, EMB), lambda i: (i, 0)),
            pl.BlockSpec((BLK, EMB), lambda i: (i, 0)),
        ],
        out_shape=[
            jax.ShapeDtypeStruct((B, EMB), jnp.float32),
            jax.ShapeDtypeStruct((B, EMB), jnp.float32),
        ],
    )(i_tag, i_qclass, emb_tag, emb_qclass)

    xs = (x_test, x_question, x_tag, x_qclass)

    xblk = pl.BlockSpec((BR, EMB), lambda i: (i, 0))
    stats = pl.pallas_call(
        _tc_stats_body,
        grid=(NB,),
        in_specs=[xblk] * 4,
        out_specs=pl.BlockSpec((2, CAT), lambda i: (0, 0)),
        out_shape=jax.ShapeDtypeStruct((2, CAT), jnp.float32),
    )(*xs)

    full = lambda *s: pl.BlockSpec(s, lambda i: (0,) * len(s))
    return pl.pallas_call(
        _tc_apply_body,
        grid=(NB,),
        in_specs=[full(2, CAT), xblk, xblk, xblk, xblk,
                  full(1, CAT), full(1, CAT), full(HID),
                  full(HID, CAT), full(HID)],
        out_specs=pl.BlockSpec((BR, HID), lambda i: (i, 0)),
        out_shape=jax.ShapeDtypeStruct((B, HID), jnp.float32),
    )(stats, *xs, bn_gamma.reshape(1, CAT), bn_beta.reshape(1, CAT),
      wn_g, wn_v, bias)
